# Initial kernel scaffold; baseline (speedup 1.0000x reference)
import jax
import jax.numpy as jnp
from jax.experimental import pallas as pl


def kernel(x, table, W, b):
    raise NotImplementedError("write your pallas kernel here")


# speedup vs baseline: 1.1165x; 1.1165x over previous
import functools

import jax
import jax.numpy as jnp
from jax import lax
from jax.experimental import pallas as pl
from jax.experimental.pallas import tpu as pltpu
from jax.experimental.pallas import tpu_sc as plsc


def _tc_body(table_ref, w_ref, b_ref, xr_ref, m2_ref, pair_ref):
    V = 10
    m = jnp.tanh(
        lax.dot_general(
            table_ref[...], w_ref[...],
            (((1,), (1,)), ((), ())),
            preferred_element_type=jnp.float32,
        )
        + b_ref[...]
    )
    p = lax.broadcasted_iota(jnp.int32, (V * V, 16), 0)
    k = lax.broadcasted_iota(jnp.int32, (V * V, 16), 1)
    onehot_a = (k == p // V).astype(jnp.float32)
    onehot_b = (k == p % V).astype(jnp.float32)
    left = jnp.dot(onehot_a, m, preferred_element_type=jnp.float32)
    right = jnp.dot(onehot_b, m, preferred_element_type=jnp.float32)
    m2_ref[...] = jnp.concatenate([left, right], axis=1)
    rr = lax.broadcasted_iota(jnp.int32, (128, 64), 0)
    cj = lax.broadcasted_iota(jnp.int32, (128, 64), 1)
    sel = ((rr == 2 * cj).astype(jnp.float32) * float(V)
           + (rr == 2 * cj + 1).astype(jnp.float32))
    r2 = lax.broadcasted_iota(jnp.int32, (64, 128), 0)
    rc = lax.broadcasted_iota(jnp.int32, (64, 128), 1)
    even_rows = (rc == 2 * r2).astype(jnp.float32)
    odd_rows = (rc == 2 * r2 + 1).astype(jnp.float32)
    xf = xr_ref[...].astype(jnp.float32)
    pe = jnp.dot(jnp.dot(even_rows, xf, preferred_element_type=jnp.float32),
                 sel, preferred_element_type=jnp.float32)
    po = jnp.dot(jnp.dot(odd_rows, xf, preferred_element_type=jnp.float32),
                 sel, preferred_element_type=jnp.float32)
    pair_ref[...] = jnp.concatenate([pe, po], axis=1).astype(jnp.int32)


@functools.lru_cache(maxsize=None)
def _make_sc_gather(n_pairs, D2):
    info = plsc.get_sparse_core_info()
    nw = info.num_cores * info.num_subcores
    pairs_per_w = n_pairs // nw
    chunk = 128
    n_chunks = pairs_per_w // chunk
    mesh = plsc.VectorSubcoreMesh(core_axis_name="c", subcore_axis_name="s")

    @functools.partial(
        pl.kernel,
        mesh=mesh,
        out_type=jax.ShapeDtypeStruct((n_pairs, D2), jnp.float32),
        scratch_types=[
            pltpu.VMEM((n_chunks, chunk), jnp.int32),
            pltpu.VMEM((pairs_per_w, D2), jnp.float32),
            pltpu.SemaphoreType.DMA,
        ],
    )
    def sc_gather(m2_hbm, pair_hbm, out_hbm, pair_v, rows_v, sem):
        wid = lax.axis_index("s") * info.num_cores + lax.axis_index("c")
        pltpu.sync_copy(pair_hbm.at[pl.ds(wid * n_chunks, n_chunks)], pair_v)
        copies = [
            pltpu.async_copy(
                m2_hbm.at[pair_v.at[c]],
                rows_v.at[pl.ds(c * chunk, chunk)],
                sem,
            )
            for c in range(n_chunks)
        ]
        for c in copies:
            c.wait()
        pltpu.sync_copy(rows_v, out_hbm.at[pl.ds(wid * pairs_per_w, pairs_per_w)])

    return sc_gather


def kernel(x, table, W, b):
    B = x.shape[0]
    V, H = table.shape
    D = W.shape[0]
    table16 = jnp.zeros((16, H), table.dtype).at[:V, :].set(table)
    xr = x.astype(jnp.int32).reshape(B // 128, 128)
    m2, pair = pl.pallas_call(
        _tc_body,
        out_shape=(
            jax.ShapeDtypeStruct((V * V, 2 * D), jnp.float32),
            jax.ShapeDtypeStruct((B // 256, 128), jnp.int32),
        ),
    )(table16, W, b.reshape(1, D), xr)
    out2 = _make_sc_gather(B // 2, 2 * D)(m2, pair)
    return out2.reshape(B, D)


# speedup vs baseline: 1.5301x; 1.3704x over previous
import functools

import jax
import jax.numpy as jnp
from jax import lax
from jax.experimental import pallas as pl
from jax.experimental.pallas import tpu as pltpu
from jax.experimental.pallas import tpu_sc as plsc


def _tc_body(table_ref, w_ref, b_ref, mt_ref):
    mt = jnp.tanh(
        lax.dot_general(
            w_ref[...], table_ref[...],
            (((1,), (1,)), ((), ())),
            preferred_element_type=jnp.float32,
        )
        + b_ref[...]
    )
    mt_ref[...] = jnp.concatenate(
        [mt, jnp.zeros((mt.shape[0], 128 - mt.shape[1]), jnp.float32)],
        axis=1,
    )


@functools.lru_cache(maxsize=None)
def _make_sc_gather(B, D):
    info = plsc.get_sparse_core_info()
    nw = info.num_cores * info.num_subcores
    per_w = B // nw
    groups = per_w // 16
    mesh = plsc.VectorSubcoreMesh(core_axis_name="c", subcore_axis_name="s")

    @functools.partial(
        pl.kernel,
        mesh=mesh,
        out_type=jax.ShapeDtypeStruct((D, B), jnp.float32),
        scratch_types=[
            pltpu.VMEM((D, 128), jnp.float32),
            pltpu.VMEM((per_w,), jnp.int32),
            pltpu.VMEM((D, per_w), jnp.float32),
            pltpu.SemaphoreType.DMA,
        ],
        compiler_params=pltpu.CompilerParams(needs_layout_passes=False),
    )
    def sc_gather(mt_hbm, idx_hbm, out_hbm, mt_v, idx_v, pout_v, sem):
        wid = lax.axis_index("s") * info.num_cores + lax.axis_index("c")
        pltpu.sync_copy(mt_hbm, mt_v)
        pltpu.sync_copy(idx_hbm.at[pl.ds(wid * per_w, per_w)], idx_v)

        def body(g, _):
            xg = idx_v[pl.ds(g * 16, 16)]
            for c in range(D):
                row = jnp.full((16,), c, jnp.int32)
                pout_v[c, pl.ds(g * 16, 16)] = plsc.load_gather(
                    mt_v, [row, xg]
                )
            return _

        lax.fori_loop(0, groups, body, None)
        pltpu.sync_copy(
            pout_v,
            out_hbm.at[pl.ds(0, D), pl.ds(wid * per_w, per_w)],
        )

    return sc_gather


def kernel(x, table, W, b):
    B = x.shape[0]
    V, H = table.shape
    D = W.shape[0]
    table16 = jnp.zeros((16, H), table.dtype).at[:V, :].set(table)
    mt = pl.pallas_call(
        _tc_body,
        out_shape=jax.ShapeDtypeStruct((D, 128), jnp.float32),
    )(table16, W, b.reshape(D, 1))
    idx = x.astype(jnp.int32)
    out_t = _make_sc_gather(B, D)(mt, idx)
    return out_t.T
